# baseline (speedup 1.0000x reference)
import functools

import numpy as np

import jax
import jax.numpy as jnp
from jax import lax
from jax.experimental import pallas as pl
from jax.experimental.pallas import tpu as pltpu
from jax.experimental.pallas import tpu_sc as plsc

N_CORES = 2
N_SUBCORES = 16
N_WORKERS = N_CORES * N_SUBCORES
LANES = 16
ROW = 128
CHUNK_E = 4096

CUTOFF = 5.0
CUTON = 3.5
A_COEF = 0.8854
A_EXP = 0.23
PHI_C = (0.18175, 0.50986, 0.28022, 0.02817)
PHI_E = (3.1998, 0.94229, 0.4029, 0.20162)


ZTAB = 64
_ZA_TABLE = (np.arange(ZTAB, dtype=np.float64) ** A_EXP).astype(np.float32)


def _scatter_slices(n_edges):
  out = []
  o = 0
  while o < n_edges:
    out.append((o, min(ROW, n_edges - o)))
    o += out[-1][1]
  return out


def _sc_kernel(e_w, a_pad, slice_w,
               an_hbm, zt_hbm, d_hbm, ii_hbm, ij_hbm, out_hbm,
               an_tab, zt_tab, d_buf, ii_buf, ij_buf, vals_buf, zbuf, accum,
               in_sem, sc_sem, tab_sem):
  cid = lax.axis_index("c")
  sid = lax.axis_index("s")
  wid = sid * N_CORES + cid
  n_full = e_w // CHUNK_E
  tail_e = e_w % CHUNK_E
  base = wid * e_w

  an_copy = pltpu.make_async_copy(an_hbm, an_tab, tab_sem)
  zt_copy = pltpu.make_async_copy(zt_hbm, zt_tab, tab_sem)
  an_copy.start()
  zt_copy.start()

  zeros16 = jnp.zeros((LANES,), jnp.float32)

  def _zero_body(k, _):
    zbuf[pl.ds(k * LANES, LANES)] = zeros16
    return _

  lax.fori_loop(0, slice_w // LANES, _zero_body, None)
  pltpu.sync_copy(zbuf, accum.at[pl.ds(sid * slice_w, slice_w)])
  an_copy.wait()
  zt_copy.wait()
  plsc.subcore_barrier()

  csum = PHI_C[0] + PHI_C[1] + PHI_C[2] + PHI_C[3]
  c0, c1, c2, c3 = (c / csum for c in PHI_C)
  e0, e1, e2, e3 = PHI_E
  inv_switch = 1.0 / (CUTOFF - CUTON)
  inv_acoef = 1.0 / A_COEF

  def _in_copies(k, b, width):
    es = pl.ds(base + k * CHUNK_E, width)
    w = pl.ds(b * CHUNK_E, width)
    return (
        pltpu.make_async_copy(d_hbm.at[es], d_buf.at[w], in_sem.at[b]),
        pltpu.make_async_copy(ii_hbm.at[es], ii_buf.at[w], in_sem.at[b]),
        pltpu.make_async_copy(ij_hbm.at[es], ij_buf.at[w], in_sem.at[b]),
    )

  def _issue_in(k, b, width=CHUNK_E):
    for c in _in_copies(k, b, width):
      c.start()

  def _wait_in(k, b, width=CHUNK_E):
    for c in _in_copies(k, b, width):
      c.wait()

  def _scatter(b, width=CHUNK_E):
    ob = b * CHUNK_E
    for o, w in _scatter_slices(width):
      pltpu.async_copy(
          vals_buf.at[pl.ds(ob + o, w)],
          accum.at[ii_buf.at[pl.ds(ob + o, w)]], sc_sem.at[b], add=True)

  def _drain_scatter(b, width=CHUNK_E):
    ob = b * CHUNK_E
    for o, w in _scatter_slices(width):
      pltpu.make_async_copy(
          vals_buf.at[pl.ds(ob + o, w)],
          accum.at[ii_buf.at[pl.ds(ob + o, w)]], sc_sem.at[b]).wait()

  def _vreg_compute(ob, v):
    sf = pl.ds(ob + v * LANES, LANES)
    ii = ii_buf[sf]
    ij = ij_buf[sf]
    d = d_buf[sf]
    ai = plsc.load_gather(an_tab, [ii])
    aj = plsc.load_gather(an_tab, [ij])
    zi = plsc.load_gather(zt_tab, [ai.astype(jnp.int32)])
    zj = plsc.load_gather(zt_tab, [aj.astype(jnp.int32)])
    x = (CUTOFF - d) * inv_switch
    poly = ((6.0 * x - 15.0) * x + 10.0) * x * x * x
    sw = jnp.where(d < CUTON, 1.0, jnp.where(d >= CUTOFF, 0.0, poly))
    t = d * (zi + zj) * inv_acoef
    phi = (c0 * jnp.exp(-e0 * t) + c1 * jnp.exp(-e1 * t)
           + c2 * jnp.exp(-e2 * t) + c3 * jnp.exp(-e3 * t))
    vals_buf[sf] = 0.5 * ai * aj * phi * sw / d

  def _compute(b, width=CHUNK_E):
    ob = b * CHUNK_E
    plsc.parallel_loop(0, width // LANES, unroll=8)(
        lambda v: _vreg_compute(ob, v))

  _issue_in(0, 0)

  def _chunk_body(k, _):
    b = lax.rem(k, 2)
    b2 = lax.rem(k + 1, 2)
    pl.when(k > 0)(lambda: _drain_scatter(b2))
    pl.when(k + 1 < n_full)(lambda: _issue_in(k + 1, b2))
    if tail_e:
      pl.when(k + 1 == n_full)(lambda: _issue_in(n_full, b2, tail_e))
    _wait_in(k, b)
    _compute(b)
    _scatter(b)
    return _

  lax.fori_loop(0, n_full, _chunk_body, None)

  if tail_e:
    bt = n_full % 2
    _wait_in(n_full, bt, tail_e)
    _compute(bt, tail_e)
    _scatter(bt, tail_e)
    _drain_scatter((n_full - 1) % 2)
    _drain_scatter(bt, tail_e)
  else:
    _drain_scatter((n_full - 1) % 2)

  plsc.subcore_barrier()

  asl = pl.ds(sid * slice_w, slice_w)
  pltpu.sync_copy(accum.at[asl], zbuf)
  pltpu.sync_copy(zbuf, out_hbm.at[pl.ds(cid * a_pad + sid * slice_w, slice_w)])


def kernel(atomic_numbers, distances, idx_i, idx_j):
  n_atoms = atomic_numbers.shape[0]
  n_edges = distances.shape[0]

  a_pad = -(-n_atoms // (N_SUBCORES * LANES)) * (N_SUBCORES * LANES)
  slice_w = a_pad // N_SUBCORES
  grain = N_WORKERS * 2 * LANES
  e_pad = -(-n_edges // grain) * grain
  e_w = e_pad // N_WORKERS

  an = jnp.pad(atomic_numbers.astype(jnp.float32), (0, a_pad - n_atoms),
               constant_values=1.0)
  zt = jnp.asarray(_ZA_TABLE)

  d = distances.astype(jnp.float32)
  ii = idx_i.astype(jnp.int32)
  ij = idx_j.astype(jnp.int32)
  if e_pad != n_edges:
    d = jnp.pad(d, (0, e_pad - n_edges), constant_values=2.0 * CUTOFF)
    ii = jnp.pad(ii, (0, e_pad - n_edges))
    ij = jnp.pad(ij, (0, e_pad - n_edges))

  mesh = plsc.VectorSubcoreMesh(core_axis_name="c", subcore_axis_name="s")
  out = pl.kernel(
      functools.partial(_sc_kernel, e_w, a_pad, slice_w),
      out_type=jax.ShapeDtypeStruct((N_CORES * a_pad,), jnp.float32),
      mesh=mesh,
      compiler_params=pltpu.CompilerParams(needs_layout_passes=False),
      scratch_types=[
          pltpu.VMEM((a_pad,), jnp.float32),
          pltpu.VMEM((ZTAB,), jnp.float32),
          pltpu.VMEM((2 * CHUNK_E,), jnp.float32),
          pltpu.VMEM((2 * CHUNK_E,), jnp.int32),
          pltpu.VMEM((2 * CHUNK_E,), jnp.int32),
          pltpu.VMEM((2 * CHUNK_E,), jnp.float32),
          pltpu.VMEM((slice_w,), jnp.float32),
          pltpu.VMEM_SHARED((a_pad,), jnp.float32),
          pltpu.SemaphoreType.DMA((2,)),
          pltpu.SemaphoreType.DMA((2,)),
          pltpu.SemaphoreType.DMA,
      ],
  )(an, zt, d, ii, ij)
  return (out[:a_pad] + out[a_pad:])[:n_atoms]


# speedup vs baseline: 1.0283x; 1.0283x over previous
import functools

import numpy as np

import jax
import jax.numpy as jnp
from jax import lax
from jax.experimental import pallas as pl
from jax.experimental.pallas import tpu as pltpu
from jax.experimental.pallas import tpu_sc as plsc

N_CORES = 2
N_SUBCORES = 16
N_WORKERS = N_CORES * N_SUBCORES
LANES = 16
ROW = 128
CHUNK_E = 2048

CUTOFF = 5.0
CUTON = 3.5
A_COEF = 0.8854
A_EXP = 0.23
PHI_C = (0.18175, 0.50986, 0.28022, 0.02817)
PHI_E = (3.1998, 0.94229, 0.4029, 0.20162)


ZTAB = 64
_ZA_TABLE = (np.arange(ZTAB, dtype=np.float64) ** A_EXP).astype(np.float32)


def _scatter_slices(n_edges):
  out = []
  o = 0
  while o < n_edges:
    out.append((o, min(ROW, n_edges - o)))
    o += out[-1][1]
  return out


def _sc_kernel(e_w, a_pad, slice_w,
               an_hbm, zt_hbm, d_hbm, ii_hbm, ij_hbm, out_hbm,
               an_tab, zt_tab, d_buf, ii_buf, ij_buf, vals_buf, zbuf, accum,
               in_sem, sc_sem, tab_sem):
  cid = lax.axis_index("c")
  sid = lax.axis_index("s")
  wid = sid * N_CORES + cid
  n_full = e_w // CHUNK_E
  tail_e = e_w % CHUNK_E
  base = wid * e_w

  an_copy = pltpu.make_async_copy(an_hbm, an_tab, tab_sem)
  zt_copy = pltpu.make_async_copy(zt_hbm, zt_tab, tab_sem)
  an_copy.start()
  zt_copy.start()

  zeros16 = jnp.zeros((LANES,), jnp.float32)

  def _zero_body(k, _):
    zbuf[pl.ds(k * LANES, LANES)] = zeros16
    return _

  lax.fori_loop(0, slice_w // LANES, _zero_body, None)
  pltpu.sync_copy(zbuf, accum.at[pl.ds(sid * slice_w, slice_w)])
  an_copy.wait()
  zt_copy.wait()
  plsc.subcore_barrier()

  csum = PHI_C[0] + PHI_C[1] + PHI_C[2] + PHI_C[3]
  c0, c1, c2, c3 = (0.5 * c / csum for c in PHI_C)
  e0, e1, e2, e3 = (e / A_COEF for e in PHI_E)
  inv_switch = 1.0 / (CUTOFF - CUTON)

  def _in_copies(k, b, width):
    es = pl.ds(base + k * CHUNK_E, width)
    w = pl.ds(b * CHUNK_E, width)
    return (
        pltpu.make_async_copy(d_hbm.at[es], d_buf.at[w], in_sem.at[b]),
        pltpu.make_async_copy(ii_hbm.at[es], ii_buf.at[w], in_sem.at[b]),
        pltpu.make_async_copy(ij_hbm.at[es], ij_buf.at[w], in_sem.at[b]),
    )

  def _issue_in(k, b, width=CHUNK_E):
    for c in _in_copies(k, b, width):
      c.start()

  def _wait_in(k, b, width=CHUNK_E):
    for c in _in_copies(k, b, width):
      c.wait()

  def _scatter(b, width=CHUNK_E):
    ob = b * CHUNK_E
    for o, w in _scatter_slices(width):
      pltpu.async_copy(
          vals_buf.at[pl.ds(ob + o, w)],
          accum.at[ii_buf.at[pl.ds(ob + o, w)]], sc_sem.at[b], add=True)

  def _drain_scatter(b, width=CHUNK_E):
    ob = b * CHUNK_E
    for o, w in _scatter_slices(width):
      pltpu.make_async_copy(
          vals_buf.at[pl.ds(ob + o, w)],
          accum.at[ii_buf.at[pl.ds(ob + o, w)]], sc_sem.at[b]).wait()

  def _vreg_compute(ob, v):
    sf = pl.ds(ob + v * LANES, LANES)
    ii = ii_buf[sf]
    ij = ij_buf[sf]
    d = d_buf[sf]
    ai = plsc.load_gather(an_tab, [ii])
    aj = plsc.load_gather(an_tab, [ij])
    zi = plsc.load_gather(zt_tab, [ai.astype(jnp.int32)])
    zj = plsc.load_gather(zt_tab, [aj.astype(jnp.int32)])
    x = (CUTOFF - d) * inv_switch
    poly = ((6.0 * x - 15.0) * x + 10.0) * x * x * x
    sw = jnp.where(d < CUTON, 1.0, jnp.where(d >= CUTOFF, 0.0, poly))
    t = d * (zi + zj)
    phi = (c0 * jnp.exp(-e0 * t) + c1 * jnp.exp(-e1 * t)
           + c2 * jnp.exp(-e2 * t) + c3 * jnp.exp(-e3 * t))
    vals_buf[sf] = ai * aj * phi * sw / d

  def _compute(b, width=CHUNK_E):
    ob = b * CHUNK_E
    plsc.parallel_loop(0, width // LANES, unroll=8)(
        lambda v: _vreg_compute(ob, v))

  _issue_in(0, 0)

  def _chunk_body(k, _):
    b = lax.rem(k, 2)
    b2 = lax.rem(k + 1, 2)
    pl.when(k > 0)(lambda: _drain_scatter(b2))
    pl.when(k + 1 < n_full)(lambda: _issue_in(k + 1, b2))
    if tail_e:
      pl.when(k + 1 == n_full)(lambda: _issue_in(n_full, b2, tail_e))
    _wait_in(k, b)
    _compute(b)
    _scatter(b)
    return _

  lax.fori_loop(0, n_full, _chunk_body, None)

  if tail_e:
    bt = n_full % 2
    _wait_in(n_full, bt, tail_e)
    _compute(bt, tail_e)
    _scatter(bt, tail_e)
    _drain_scatter((n_full - 1) % 2)
    _drain_scatter(bt, tail_e)
  else:
    _drain_scatter((n_full - 1) % 2)

  plsc.subcore_barrier()

  asl = pl.ds(sid * slice_w, slice_w)
  pltpu.sync_copy(accum.at[asl], zbuf)
  pltpu.sync_copy(zbuf, out_hbm.at[pl.ds(cid * a_pad + sid * slice_w, slice_w)])


def kernel(atomic_numbers, distances, idx_i, idx_j):
  n_atoms = atomic_numbers.shape[0]
  n_edges = distances.shape[0]

  a_pad = -(-n_atoms // (N_SUBCORES * LANES)) * (N_SUBCORES * LANES)
  slice_w = a_pad // N_SUBCORES
  grain = N_WORKERS * 2 * LANES
  e_pad = -(-n_edges // grain) * grain
  e_w = e_pad // N_WORKERS

  an = jnp.pad(atomic_numbers.astype(jnp.float32), (0, a_pad - n_atoms),
               constant_values=1.0)
  zt = jnp.asarray(_ZA_TABLE)

  d = distances.astype(jnp.float32)
  ii = idx_i.astype(jnp.int32)
  ij = idx_j.astype(jnp.int32)
  if e_pad != n_edges:
    d = jnp.pad(d, (0, e_pad - n_edges), constant_values=2.0 * CUTOFF)
    ii = jnp.pad(ii, (0, e_pad - n_edges))
    ij = jnp.pad(ij, (0, e_pad - n_edges))

  mesh = plsc.VectorSubcoreMesh(core_axis_name="c", subcore_axis_name="s")
  out = pl.kernel(
      functools.partial(_sc_kernel, e_w, a_pad, slice_w),
      out_type=jax.ShapeDtypeStruct((N_CORES * a_pad,), jnp.float32),
      mesh=mesh,
      compiler_params=pltpu.CompilerParams(needs_layout_passes=False),
      scratch_types=[
          pltpu.VMEM((a_pad,), jnp.float32),
          pltpu.VMEM((ZTAB,), jnp.float32),
          pltpu.VMEM((2 * CHUNK_E,), jnp.float32),
          pltpu.VMEM((2 * CHUNK_E,), jnp.int32),
          pltpu.VMEM((2 * CHUNK_E,), jnp.int32),
          pltpu.VMEM((2 * CHUNK_E,), jnp.float32),
          pltpu.VMEM((slice_w,), jnp.float32),
          pltpu.VMEM_SHARED((a_pad,), jnp.float32),
          pltpu.SemaphoreType.DMA((2,)),
          pltpu.SemaphoreType.DMA((2,)),
          pltpu.SemaphoreType.DMA,
      ],
  )(an, zt, d, ii, ij)
  return (out[:a_pad] + out[a_pad:])[:n_atoms]
